# baseline (speedup 1.0000x reference)
import functools

import jax
import jax.numpy as jnp
import numpy as np
from jax import lax
from jax.experimental import pallas as pl
from jax.experimental.pallas import tpu as pltpu
from jax.experimental.pallas import tpu_sc as plsc

_MAX_SEQ_LEN = 4096


def _position_enc_table(max_seq_len: int, dim: int) -> jnp.ndarray:
    pos = np.arange(max_seq_len, dtype=np.float64)[:, None]
    i = np.arange(dim, dtype=np.float64)[None, :]
    enc = pos / np.power(10000.0, (i - (i % 2)) / dim)
    enc[:, 0::2] = np.sin(enc[:, 0::2])
    enc[:, 1::2] = np.cos(enc[:, 1::2])
    return jnp.asarray(enc, dtype=jnp.float32)


def _broadcast_rows(table, b, t, dim):
    info = plsc.get_sparse_core_info()
    nw = info.num_cores * info.num_subcores
    rows_per_w = t // nw
    chunk = min(rows_per_w, 64)
    n_chunks = rows_per_w // chunk
    mesh = plsc.VectorSubcoreMesh(core_axis_name="c", subcore_axis_name="s")

    @functools.partial(
        pl.kernel,
        mesh=mesh,
        out_type=jax.ShapeDtypeStruct((b * t, dim), jnp.float32),
        scratch_types=[
            pltpu.VMEM((chunk, dim), jnp.float32),
            pltpu.SemaphoreType.DMA,
        ],
    )
    def k(table_hbm, out_hbm, buf, sem):
        wid = lax.axis_index("s") * info.num_cores + lax.axis_index("c")
        base = wid * rows_per_w
        for c in range(n_chunks):
            row0 = base + c * chunk
            pltpu.sync_copy(table_hbm.at[pl.ds(row0, chunk)], buf)
            for bb in range(b):
                pltpu.sync_copy(buf, out_hbm.at[pl.ds(bb * t + row0, chunk)])

    return k(table).reshape(b, t, dim)


def kernel(inputs):
    b, t, dim = inputs.shape
    table = _position_enc_table(_MAX_SEQ_LEN, dim)[:t]
    return _broadcast_rows(table, b, t, dim)
